# baseline (speedup 1.0000x reference)
import functools

import jax
import jax.numpy as jnp
from jax import lax
from jax.experimental import pallas as pl
from jax.experimental.pallas import tpu as pltpu
from jax.experimental.pallas import tpu_sc as plsc

_NC = 2
_NS = 16
_NW = _NC * _NS
_LANES = 16


@functools.cache
def _build_tc_index(S, D0, D1, D2, G=64):
    n_blocks = S // 128

    def tc_index(x_ref, idx_ref):
        xb = x_ref[...]
        x0 = xb[:, 0, :]
        x1 = xb[:, 1, :]
        x2 = xb[:, 2, :]
        idx_ref[...] = (
            x0 * (D1 * D2)
            + (x1 >> 3) * (8 * D2)
            + (x2 >> 7) * 1024
            + (x1 & 7) * 128
            + (x2 & 127)
        )

    return pl.pallas_call(
        tc_index,
        grid=(n_blocks // G,),
        in_specs=[pl.BlockSpec((G, 3, 128), lambda g: (g, 0, 0))],
        out_specs=pl.BlockSpec((G, 128), lambda g: (g, 0)),
        out_shape=jax.ShapeDtypeStruct((n_blocks, 128), jnp.int32),
    )


@functools.cache
def _build_sc_gather(S):
    b_per_w = S // _NW
    C = min(b_per_w, 8192)
    n_chunks = b_per_w // C

    mesh = plsc.VectorSubcoreMesh(core_axis_name="c", subcore_axis_name="s")

    @functools.partial(
        pl.kernel,
        mesh=mesh,
        compiler_params=pltpu.CompilerParams(needs_layout_passes=False),
        out_type=jax.ShapeDtypeStruct((S,), jnp.float32),
        scratch_types=[
            pltpu.VMEM((C,), jnp.int32),
            pltpu.VMEM((C,), jnp.int32),
            pltpu.VMEM((C,), jnp.float32),
            pltpu.VMEM((C,), jnp.float32),
            pltpu.SemaphoreType.DMA,
            pltpu.SemaphoreType.DMA,
            pltpu.SemaphoreType.DMA,
            pltpu.SemaphoreType.DMA,
            pltpu.SemaphoreType.DMA,
            pltpu.SemaphoreType.DMA,
        ],
    )
    def sc_gather(idx_hbm, w_hbm, out_hbm,
                  idx_a, idx_b, out_a, out_b,
                  si_a, si_b, sg_a, sg_b, so_a, so_b):
        wid = lax.axis_index("s") * _NC + lax.axis_index("c")
        base = wid * b_per_w
        idxv = (idx_a, idx_b)
        outv = (out_a, out_b)
        si = (si_a, si_b)
        sg = (sg_a, sg_b)
        so = (so_a, so_b)

        def start_in(i):
            off = base + i * C
            b = i & 1
            return pltpu.async_copy(idx_hbm.at[pl.ds(off, C)], idxv[b], si[b])

        ins = [None] * n_chunks
        gats = [None] * n_chunks
        outs = [None] * n_chunks
        ins[0] = start_in(0)
        for i in range(n_chunks):
            b = i & 1
            if i + 1 < n_chunks:
                ins[i + 1] = start_in(i + 1)
            ins[i].wait()
            if i >= 2:
                outs[i - 2].wait()
            gats[i] = pltpu.async_copy(w_hbm.at[idxv[b]], outv[b], sg[b])
            if i >= 1:
                gats[i - 1].wait()
                off_p = base + (i - 1) * C
                outs[i - 1] = pltpu.async_copy(
                    outv[b ^ 1], out_hbm.at[pl.ds(off_p, C)], so[b ^ 1])
        last = n_chunks - 1
        bl = last & 1
        gats[last].wait()
        outs[last] = pltpu.async_copy(
            outv[bl], out_hbm.at[pl.ds(base + last * C, C)], so[bl])
        if n_chunks >= 2:
            outs[last - 1].wait()
        outs[last].wait()

    return sc_gather


def kernel(x, weights):
    S = x.shape[0]
    D0, D1, D2 = weights.shape
    w_phys = (
        weights.reshape(D0, D1 // 8, 8, D2 // 128, 128)
        .transpose(0, 1, 3, 2, 4)
        .reshape(D0 * D1 * D2)
    )
    x_blocks = x.reshape(S // 128, 128, 3).transpose(0, 2, 1)
    idx = _build_tc_index(S, D0, D1, D2)(x_blocks).reshape(S)
    return _build_sc_gather(S)(idx, w_phys)


# speedup vs baseline: 1.6574x; 1.6574x over previous
import functools

import jax
import jax.numpy as jnp
from jax import lax
from jax.experimental import pallas as pl
from jax.experimental.pallas import tpu as pltpu
from jax.experimental.pallas import tpu_sc as plsc

_NC = 2
_NS = 16
_NW = _NC * _NS
_LANES = 16


@functools.cache
def _build_sc_gather(S, D0, D1, D2):
    b_per_w = S // _NW
    C = min(b_per_w, 8192)
    n_chunks = b_per_w // C

    mesh = plsc.VectorSubcoreMesh(core_axis_name="c", subcore_axis_name="s")

    @functools.partial(
        pl.kernel,
        mesh=mesh,
        compiler_params=pltpu.CompilerParams(needs_layout_passes=False),
        out_type=jax.ShapeDtypeStruct((S,), jnp.float32),
        scratch_types=[
            pltpu.VMEM((4 * C,), jnp.int32),
            pltpu.VMEM((4 * C,), jnp.int32),
            pltpu.VMEM((C,), jnp.int32),
            pltpu.VMEM((C,), jnp.int32),
            pltpu.VMEM((C,), jnp.float32),
            pltpu.VMEM((C,), jnp.float32),
            pltpu.SemaphoreType.DMA,
            pltpu.SemaphoreType.DMA,
            pltpu.SemaphoreType.DMA,
            pltpu.SemaphoreType.DMA,
            pltpu.SemaphoreType.DMA,
            pltpu.SemaphoreType.DMA,
        ],
    )
    def sc_gather(x_hbm, w_hbm, out_hbm,
                  xin_a, xin_b, idx_a, idx_b, out_a, out_b,
                  si_a, si_b, sg_a, sg_b, so_a, so_b):
        wid = lax.axis_index("s") * _NC + lax.axis_index("c")
        base = wid * b_per_w
        xinv = (xin_a, xin_b)
        idxv = (idx_a, idx_b)
        outv = (out_a, out_b)
        si = (si_a, si_b)
        sg = (sg_a, sg_b)
        so = (so_a, so_b)

        def start_in(i):
            off = 4 * (base + i * C)
            b = i & 1
            return pltpu.async_copy(x_hbm.at[pl.ds(off, 4 * C)], xinv[b], si[b])

        def compute(i):
            b = i & 1
            xin = xinv[b]

            def grp_body(g, c):
                for r in range(8):
                    off = g * 512 + r * _LANES
                    sl = pl.ds((g * 8 + r) * _LANES, _LANES)
                    x0 = xin[pl.ds(off, _LANES)]
                    x1 = xin[pl.ds(off + 128, _LANES)]
                    x2 = xin[pl.ds(off + 256, _LANES)]
                    idxv[b][sl] = (
                        x0 * (D1 * D2)
                        + (x1 >> 3) * (8 * D2)
                        + (x2 >> 7) * 1024
                        + (x1 & 7) * 128
                        + (x2 & 127)
                    )
                return c

            lax.fori_loop(0, C // 128, grp_body, 0)

        ins = [None] * n_chunks
        gats = [None] * n_chunks
        outs = [None] * n_chunks
        ins[0] = start_in(0)
        for i in range(n_chunks):
            b = i & 1
            if i + 1 < n_chunks:
                ins[i + 1] = start_in(i + 1)
            ins[i].wait()
            compute(i)
            if i >= 2:
                outs[i - 2].wait()
            gats[i] = pltpu.async_copy(w_hbm.at[idxv[b]], outv[b], sg[b])
            if i >= 1:
                gats[i - 1].wait()
                off_p = base + (i - 1) * C
                outs[i - 1] = pltpu.async_copy(
                    outv[b ^ 1], out_hbm.at[pl.ds(off_p, C)], so[b ^ 1])
        last = n_chunks - 1
        bl = last & 1
        gats[last].wait()
        outs[last] = pltpu.async_copy(
            outv[bl], out_hbm.at[pl.ds(base + last * C, C)], so[bl])
        if n_chunks >= 2:
            outs[last - 1].wait()
        outs[last].wait()

    return sc_gather


def kernel(x, weights):
    S = x.shape[0]
    D0, D1, D2 = weights.shape
    w_phys = (
        weights.reshape(D0, D1 // 8, 8, D2 // 128, 128)
        .transpose(0, 1, 3, 2, 4)
        .reshape(D0 * D1 * D2)
    )
    x_phys = (
        jnp.pad(x, ((0, 0), (0, 1)))
        .reshape(S // 128, 128, 4)
        .transpose(0, 2, 1)
        .reshape(4 * S)
    )
    return _build_sc_gather(S, D0, D1, D2)(x_phys, w_phys)


# speedup vs baseline: 1.6836x; 1.0159x over previous
import functools

import jax
import jax.numpy as jnp
from jax import lax
from jax.experimental import pallas as pl
from jax.experimental.pallas import tpu as pltpu
from jax.experimental.pallas import tpu_sc as plsc

_NC = 2
_NS = 16
_NW = _NC * _NS
_LANES = 16


@functools.cache
def _build_sc_gather(S, D0, D1, D2):
    b_per_w = S // _NW
    C = min(b_per_w, 4096)
    n_chunks = b_per_w // C

    mesh = plsc.VectorSubcoreMesh(core_axis_name="c", subcore_axis_name="s")

    @functools.partial(
        pl.kernel,
        mesh=mesh,
        compiler_params=pltpu.CompilerParams(needs_layout_passes=False),
        out_type=jax.ShapeDtypeStruct((S,), jnp.float32),
        scratch_types=[
            pltpu.VMEM((4 * C,), jnp.int32),
            pltpu.VMEM((4 * C,), jnp.int32),
            pltpu.VMEM((C,), jnp.int32),
            pltpu.VMEM((C,), jnp.int32),
            pltpu.VMEM((C,), jnp.float32),
            pltpu.VMEM((C,), jnp.float32),
            pltpu.SemaphoreType.DMA,
            pltpu.SemaphoreType.DMA,
            pltpu.SemaphoreType.DMA,
            pltpu.SemaphoreType.DMA,
            pltpu.SemaphoreType.DMA,
            pltpu.SemaphoreType.DMA,
        ],
    )
    def sc_gather(x_hbm, w_hbm, out_hbm,
                  xin_a, xin_b, idx_a, idx_b, out_a, out_b,
                  si_a, si_b, sg_a, sg_b, so_a, so_b):
        wid = lax.axis_index("s") * _NC + lax.axis_index("c")
        base = wid * b_per_w
        xinv = (xin_a, xin_b)
        idxv = (idx_a, idx_b)
        outv = (out_a, out_b)
        si = (si_a, si_b)
        sg = (sg_a, sg_b)
        so = (so_a, so_b)

        def start_in(i):
            off = 4 * (base + i * C)
            b = i & 1
            return pltpu.async_copy(x_hbm.at[pl.ds(off, 4 * C)], xinv[b], si[b])

        def compute(i):
            b = i & 1
            xin = xinv[b]

            def grp_body(g, c):
                for r in range(8):
                    off = g * 512 + r * _LANES
                    sl = pl.ds((g * 8 + r) * _LANES, _LANES)
                    x0 = xin[pl.ds(off, _LANES)]
                    x1 = xin[pl.ds(off + 128, _LANES)]
                    x2 = xin[pl.ds(off + 256, _LANES)]
                    idxv[b][sl] = (
                        x0 * (D1 * D2)
                        + (x1 >> 3) * (8 * D2)
                        + (x2 >> 7) * 1024
                        + (x1 & 7) * 128
                        + (x2 & 127)
                    )
                return c

            lax.fori_loop(0, C // 128, grp_body, 0)

        ins = [None] * n_chunks
        gats = [None] * n_chunks
        outs = [None] * n_chunks
        ins[0] = start_in(0)
        for i in range(n_chunks):
            b = i & 1
            if i + 1 < n_chunks:
                ins[i + 1] = start_in(i + 1)
            ins[i].wait()
            compute(i)
            if i >= 2:
                outs[i - 2].wait()
            gats[i] = pltpu.async_copy(w_hbm.at[idxv[b]], outv[b], sg[b])
            if i >= 1:
                gats[i - 1].wait()
                off_p = base + (i - 1) * C
                outs[i - 1] = pltpu.async_copy(
                    outv[b ^ 1], out_hbm.at[pl.ds(off_p, C)], so[b ^ 1])
        last = n_chunks - 1
        bl = last & 1
        gats[last].wait()
        outs[last] = pltpu.async_copy(
            outv[bl], out_hbm.at[pl.ds(base + last * C, C)], so[bl])
        if n_chunks >= 2:
            outs[last - 1].wait()
        outs[last].wait()

    return sc_gather


def kernel(x, weights):
    S = x.shape[0]
    D0, D1, D2 = weights.shape
    w_phys = (
        weights.reshape(D0, D1 // 8, 8, D2 // 128, 128)
        .transpose(0, 1, 3, 2, 4)
        .reshape(D0 * D1 * D2)
    )
    x_phys = (
        jnp.pad(x, ((0, 0), (0, 1)))
        .reshape(S // 128, 128, 4)
        .transpose(0, 2, 1)
        .reshape(4 * S)
    )
    return _build_sc_gather(S, D0, D1, D2)(x_phys, w_phys)
